# baseline (speedup 1.0000x reference)
import functools

import jax
import jax.numpy as jnp
from jax import lax
from jax.experimental import pallas as pl
from jax.experimental.pallas import tpu as pltpu
from jax.experimental.pallas import tpu_sc as plsc

BATCH = 16384
EMBED_DIM = 64
NUM_ROWS = 1000000

_NUM_CORES = 2
_NUM_SUBCORES = 16
_NUM_WORKERS = _NUM_CORES * _NUM_SUBCORES
_B_PER_W = BATCH // _NUM_WORKERS
_GROUPS = _B_PER_W // 16

_SLAB_W = 128
_NBUF = 8


@functools.cache
def _make_sc_gather():
    mesh = plsc.VectorSubcoreMesh(core_axis_name="c", subcore_axis_name="s")

    @functools.partial(
        pl.kernel,
        out_type=jax.ShapeDtypeStruct((BATCH, 2 * EMBED_DIM), jnp.float32),
        mesh=mesh,
        scratch_types=[
            pltpu.VMEM((_B_PER_W,), jnp.int32),
            pltpu.VMEM((_B_PER_W,), jnp.int32),
            pltpu.VMEM((_NBUF, EMBED_DIM, _SLAB_W), jnp.float32),
            pltpu.VMEM((2, 16, 2 * EMBED_DIM), jnp.float32),
            [pltpu.SemaphoreType.DMA] * _NBUF,
            [pltpu.SemaphoreType.DMA] * 2,
        ],
        compiler_params=pltpu.CompilerParams(
            use_tc_tiling_on_sc=True, needs_layout_passes=False
        ),
    )
    def _sc_gather(
        utabT_hbm,
        itabT_hbm,
        uid_hbm,
        iid_hbm,
        out_hbm,
        uidx_v,
        iidx_v,
        ring_v,
        stage_v,
        sems,
        flush_sems,
    ):
        wid = lax.axis_index("s") * _NUM_CORES + lax.axis_index("c")
        base = wid * _B_PER_W
        pltpu.sync_copy(uid_hbm.at[pl.ds(base, _B_PER_W)], uidx_v)
        pltpu.sync_copy(iid_hbm.at[pl.ds(base, _B_PER_W)], iidx_v)

        rows = [lax.iota(jnp.int32, 16) + 16 * k for k in range(4)]

        def issue(tab, c, slot):
            start = pl.multiple_of((c // _SLAB_W) * _SLAB_W, _SLAB_W)
            pltpu.async_copy(
                tab.at[:, pl.ds(start, _SLAB_W)], ring_v.at[slot], sems[slot]
            )

        def do_group(tab, cur, nxt_tab, nxt_vec, tmod, col0, guard_last):
            for l in range(16):
                slot = l % _NBUF
                pltpu.make_async_copy(
                    tab.at[:, pl.ds(0, _SLAB_W)], ring_v.at[slot], sems[slot]
                ).wait()
                cm = cur[l] % _SLAB_W
                cmv = jnp.full((16,), cm, dtype=jnp.int32)
                for k in range(4):
                    val = plsc.load_gather(ring_v.at[slot], [rows[k], cmv])
                    stage_v[tmod, l, pl.ds(col0 + 16 * k, 16)] = val

                la = l + _NBUF
                c_la = cur[la] if la < 16 else nxt_vec[la - 16]
                i_tab = tab if la < 16 else nxt_tab
                if guard_last is None or la < 16:
                    issue(i_tab, c_la, slot)
                else:
                    @pl.when(guard_last)
                    def _():
                        issue(i_tab, c_la, slot)

        def one_group(t, tmod):
            sel = t * 16
            nsel = jnp.minimum(t + 1, _GROUPS - 1) * 16
            u_cur = uidx_v[pl.ds(sel, 16)]
            i_cur = iidx_v[pl.ds(sel, 16)]
            u_nxt = uidx_v[pl.ds(nsel, 16)]

            @pl.when(t >= 2)
            def _():
                pltpu.make_async_copy(
                    stage_v.at[tmod],
                    out_hbm.at[pl.ds(base + (t - 2) * 16, 16)],
                    flush_sems[tmod],
                ).wait()

            do_group(utabT_hbm, u_cur, itabT_hbm, i_cur, tmod, 0, None)
            do_group(itabT_hbm, i_cur, utabT_hbm, u_nxt, tmod,
                     EMBED_DIM, t < _GROUPS - 1)

            pltpu.async_copy(
                stage_v.at[tmod],
                out_hbm.at[pl.ds(base + t * 16, 16)],
                flush_sems[tmod],
            )

        def body(p, _):
            one_group(p * 2, 0)
            one_group(p * 2 + 1, 1)
            return ()

        u0 = uidx_v[pl.ds(0, 16)]
        for l in range(_NBUF):
            issue(utabT_hbm, u0[l], l)

        lax.fori_loop(0, _GROUPS // 2, body, (), unroll=False)

        for tmod, t in ((( _GROUPS - 2) % 2, _GROUPS - 2),
                        ((_GROUPS - 1) % 2, _GROUPS - 1)):
            pltpu.make_async_copy(
                stage_v.at[tmod],
                out_hbm.at[pl.ds(base + t * 16, 16)],
                flush_sems[tmod],
            ).wait()

    return _sc_gather


_MLP_BLOCK = 2048


def _mlp_body(x_ref, w1_ref, b1_ref, w2_ref, b2_ref, w3_ref, b3_ref, out_ref):
    h = jnp.dot(x_ref[...], w1_ref[...], preferred_element_type=jnp.float32)
    h = jnp.maximum(h + b1_ref[...], 0.0)
    h = jnp.dot(h, w2_ref[...], preferred_element_type=jnp.float32)
    h = jnp.maximum(h + b2_ref[...], 0.0)
    out_ref[...] = (
        jnp.dot(h, w3_ref[...], preferred_element_type=jnp.float32)
        + b3_ref[...]
    )


def _mlp(x, W1, b1, W2, b2, W3, b3):
    grid = (BATCH // _MLP_BLOCK,)
    full = lambda g: (0, 0)
    return pl.pallas_call(
        _mlp_body,
        grid=grid,
        in_specs=[
            pl.BlockSpec((_MLP_BLOCK, 2 * EMBED_DIM), lambda g: (g, 0)),
            pl.BlockSpec((2 * EMBED_DIM, 64), full),
            pl.BlockSpec((1, 64), full),
            pl.BlockSpec((64, 32), full),
            pl.BlockSpec((1, 32), full),
            pl.BlockSpec((32, 1), full),
            pl.BlockSpec((1, 1), full),
        ],
        out_specs=pl.BlockSpec((_MLP_BLOCK, 1), lambda g: (g, 0)),
        out_shape=jax.ShapeDtypeStruct((BATCH, 1), jnp.float32),
    )(x, W1, b1.reshape(1, 64), W2, b2.reshape(1, 32), W3, b3.reshape(1, 1))


def kernel(user_id, item_id, user_table, item_table, W1, b1, W2, b2, W3, b3):
    emb = _make_sc_gather()(
        user_table.T, item_table.T,
        user_id.astype(jnp.int32), item_id.astype(jnp.int32),
    )
    return _mlp(emb, W1, b1, W2, b2, W3, b3)


# speedup vs baseline: 1.0130x; 1.0130x over previous
import functools

import jax
import jax.numpy as jnp
from jax import lax
from jax.experimental import pallas as pl
from jax.experimental.pallas import tpu as pltpu
from jax.experimental.pallas import tpu_sc as plsc

BATCH = 16384
EMBED_DIM = 64
NUM_ROWS = 1000000

_NUM_CORES = 2
_NUM_SUBCORES = 16
_NUM_WORKERS = _NUM_CORES * _NUM_SUBCORES
_B_PER_W = BATCH // _NUM_WORKERS
_GROUPS = _B_PER_W // 16

_SLAB_W = 128
_NBUF = 8


@functools.cache
def _make_sc_gather():
    mesh = plsc.VectorSubcoreMesh(core_axis_name="c", subcore_axis_name="s")

    @functools.partial(
        pl.kernel,
        out_type=jax.ShapeDtypeStruct((BATCH, 2 * EMBED_DIM), jnp.float32),
        mesh=mesh,
        scratch_types=[
            pltpu.VMEM((_B_PER_W,), jnp.int32),
            pltpu.VMEM((_B_PER_W,), jnp.int32),
            pltpu.VMEM((_NBUF, EMBED_DIM, _SLAB_W), jnp.float32),
            pltpu.VMEM((2, 16, 2 * EMBED_DIM), jnp.float32),
            [pltpu.SemaphoreType.DMA] * _NBUF,
            [pltpu.SemaphoreType.DMA] * 2,
        ],
        compiler_params=pltpu.CompilerParams(
            use_tc_tiling_on_sc=True, needs_layout_passes=False
        ),
    )
    def _sc_gather(
        utabT_hbm,
        itabT_hbm,
        uid_hbm,
        iid_hbm,
        out_hbm,
        uidx_v,
        iidx_v,
        ring_v,
        stage_v,
        sems,
        flush_sems,
    ):
        wid = lax.axis_index("s") * _NUM_CORES + lax.axis_index("c")
        base = wid * _B_PER_W
        pltpu.sync_copy(uid_hbm.at[pl.ds(base, _B_PER_W)], uidx_v)
        pltpu.sync_copy(iid_hbm.at[pl.ds(base, _B_PER_W)], iidx_v)

        rows = [lax.iota(jnp.int32, 16) + 16 * k for k in range(4)]

        def issue(tab, c, slot):
            start = pl.multiple_of((c // _SLAB_W) * _SLAB_W, _SLAB_W)
            pltpu.async_copy(
                tab.at[:, pl.ds(start, _SLAB_W)], ring_v.at[slot], sems[slot]
            )

        def do_group(tab, cur, nxt_tab, nxt_vec, tmod, col0, guard_last):
            for l in range(16):
                slot = l % _NBUF
                pltpu.make_async_copy(
                    tab.at[:, pl.ds(0, _SLAB_W)], ring_v.at[slot], sems[slot]
                ).wait()
                cm = cur[l] % _SLAB_W
                cmv = jnp.full((16,), cm, dtype=jnp.int32)
                for k in range(4):
                    val = plsc.load_gather(ring_v.at[slot], [rows[k], cmv])
                    stage_v[tmod, l, pl.ds(col0 + 16 * k, 16)] = val

                la = l + _NBUF
                c_la = cur[la] if la < 16 else nxt_vec[la - 16]
                i_tab = tab if la < 16 else nxt_tab
                if guard_last is None or la < 16:
                    issue(i_tab, c_la, slot)
                else:
                    @pl.when(guard_last)
                    def _():
                        issue(i_tab, c_la, slot)

        def one_group(t, tmod):
            sel = t * 16
            nsel = jnp.minimum(t + 1, _GROUPS - 1) * 16
            u_cur = uidx_v[pl.ds(sel, 16)]
            i_cur = iidx_v[pl.ds(sel, 16)]
            u_nxt = uidx_v[pl.ds(nsel, 16)]

            @pl.when(t >= 2)
            def _():
                pltpu.make_async_copy(
                    stage_v.at[tmod],
                    out_hbm.at[pl.ds(base + (t - 2) * 16, 16)],
                    flush_sems[tmod],
                ).wait()

            do_group(utabT_hbm, u_cur, itabT_hbm, i_cur, tmod, 0, None)
            do_group(itabT_hbm, i_cur, utabT_hbm, u_nxt, tmod,
                     EMBED_DIM, t < _GROUPS - 1)

            pltpu.async_copy(
                stage_v.at[tmod],
                out_hbm.at[pl.ds(base + t * 16, 16)],
                flush_sems[tmod],
            )

        def body(p, _):
            one_group(p * 2, 0)
            one_group(p * 2 + 1, 1)
            return ()

        u0 = uidx_v[pl.ds(0, 16)]
        for l in range(_NBUF):
            issue(utabT_hbm, u0[l], l)

        lax.fori_loop(0, _GROUPS // 2, body, (), unroll=False)

        for tmod, t in ((( _GROUPS - 2) % 2, _GROUPS - 2),
                        ((_GROUPS - 1) % 2, _GROUPS - 1)):
            pltpu.make_async_copy(
                stage_v.at[tmod],
                out_hbm.at[pl.ds(base + t * 16, 16)],
                flush_sems[tmod],
            ).wait()

    return _sc_gather


_MLP_BLOCK = 8192


def _mlp_body(x_ref, w1_ref, b1_ref, w2_ref, b2_ref, w3_ref, b3_ref, out_ref):
    h = jnp.dot(x_ref[...], w1_ref[...], preferred_element_type=jnp.float32)
    h = jnp.maximum(h + b1_ref[...], 0.0)
    h = jnp.dot(h, w2_ref[...], preferred_element_type=jnp.float32)
    h = jnp.maximum(h + b2_ref[...], 0.0)
    out_ref[...] = (
        jnp.dot(h, w3_ref[...], preferred_element_type=jnp.float32)
        + b3_ref[...]
    )


def _mlp(x, W1, b1, W2, b2, W3, b3):
    grid = (BATCH // _MLP_BLOCK,)
    full = lambda g: (0, 0)
    return pl.pallas_call(
        _mlp_body,
        grid=grid,
        in_specs=[
            pl.BlockSpec((_MLP_BLOCK, 2 * EMBED_DIM), lambda g: (g, 0)),
            pl.BlockSpec((2 * EMBED_DIM, 64), full),
            pl.BlockSpec((1, 64), full),
            pl.BlockSpec((64, 32), full),
            pl.BlockSpec((1, 32), full),
            pl.BlockSpec((32, 1), full),
            pl.BlockSpec((1, 1), full),
        ],
        out_specs=pl.BlockSpec((_MLP_BLOCK, 1), lambda g: (g, 0)),
        out_shape=jax.ShapeDtypeStruct((BATCH, 1), jnp.float32),
    )(x, W1, b1.reshape(1, 64), W2, b2.reshape(1, 32), W3, b3.reshape(1, 1))


def kernel(user_id, item_id, user_table, item_table, W1, b1, W2, b2, W3, b3):
    emb = _make_sc_gather()(
        user_table.T, item_table.T,
        user_id.astype(jnp.int32), item_id.astype(jnp.int32),
    )
    return _mlp(emb, W1, b1, W2, b2, W3, b3)
